# Initial kernel scaffold; baseline (speedup 1.0000x reference)
import jax
import jax.numpy as jnp
from jax.experimental import pallas as pl


def kernel(traj_feat, traj_len, adj_feat, route_choice_travel_progress, route_choice_angle, route_choice_trans_prob, route_choice_selected_mask, route_choice_unselected_mask, Wtp, btp, Wang, bang, Wtr, btr, Wq, bq, Wg, bg, eW1, eb1, eW2, eb2, eW3, eb3, sW1, sb1, sW2, sb2, sW3, sb3, fW1, fb1, fW2, fb2, fW3, fb3):
    raise NotImplementedError("write your pallas kernel here")


# speedup vs baseline: 9.3637x; 9.3637x over previous
import math

import jax
import jax.numpy as jnp
from jax.experimental import pallas as pl
from jax.experimental.pallas import tpu as pltpu

B, T1, T2, C = 2, 512, 4, 128
E, K = 64, 2
NT = B * T1
NR = NT * T2
G = 32
NSLOT = 4096
NTILE = NSLOT // G
NEG = -1e30


def _sig(x):
    return 1.0 / (1.0 + jnp.exp(-x))


def _gate_route_kernel(cif_ref, wg_ref, bg_ref, mask_ref,
                       vio_ref, te_ref, st_ref, sw_ref):
    g = jnp.dot(cif_ref[...], wg_ref[...],
                preferred_element_type=jnp.float32) + bg_ref[...]
    probs = _sig(g)
    lane = jax.lax.broadcasted_iota(jnp.int32, (NT, E), 1)
    m0 = jnp.max(g, axis=1, keepdims=True)
    idx0 = jnp.min(jnp.where(g >= m0, lane, E), axis=1, keepdims=True)
    oh0 = lane == idx0
    g1 = jnp.where(oh0, NEG, g)
    m1 = jnp.max(g1, axis=1, keepdims=True)
    idx1 = jnp.min(jnp.where(g1 >= m1, lane, E), axis=1, keepdims=True)
    oh1 = lane == idx1
    oh0f = oh0.astype(jnp.float32)
    oh1f = oh1.astype(jnp.float32)
    p0 = jnp.sum(probs * oh0f, axis=1, keepdims=True)
    p1 = jnp.sum(probs * oh1f, axis=1, keepdims=True)
    w0 = p0 / (p0 + p1)
    w1 = p1 / (p0 + p1)

    counts_m = jnp.sum(mask_ref[...] * (oh0f + oh1f), axis=0, keepdims=True)
    mean = jnp.sum(counts_m, axis=1, keepdims=True) * (1.0 / E)
    mx = jnp.max(counts_m, axis=1, keepdims=True)
    vio_ref[...] = (mx - mean) / (mean + 1e-5)

    s_tok = oh0f + oh1f
    c = s_tok
    sh = 1
    while sh < NT:
        c = c + jnp.concatenate(
            [jnp.zeros((sh, E), jnp.float32), c[:NT - sh]], axis=0)
        sh *= 2
    excl = c - s_tok
    counts = jnp.sum(s_tok, axis=0, keepdims=True)
    padded = ((counts.astype(jnp.int32) + (G - 1)) // G) * G
    r64 = jax.lax.broadcasted_iota(jnp.int32, (E, E), 0)
    c64 = jax.lax.broadcasted_iota(jnp.int32, (E, E), 1)
    lt = (r64 < c64).astype(jnp.float32)
    off = jnp.dot(padded.astype(jnp.float32), lt,
                  preferred_element_type=jnp.float32)

    rank0 = jnp.sum(excl * oh0f, axis=1, keepdims=True)
    rank1 = jnp.sum(excl * oh1f, axis=1, keepdims=True)
    dst0 = jnp.sum(off * oh0f, axis=1, keepdims=True) + rank0
    dst1 = jnp.sum(off * oh1f, axis=1, keepdims=True) + rank1
    tokcol = jax.lax.broadcasted_iota(jnp.int32, (NT, 1), 0).astype(jnp.float32)
    dn = (((0,), (0,)), ((), ()))
    CH = 1024
    for ch in range(NSLOT // CH):
        prow = (jax.lax.broadcasted_iota(jnp.int32, (1, CH), 1)
                + (CH * ch)).astype(jnp.float32)
        m0c = (dst0 == prow).astype(jnp.float32)
        m1c = (dst1 == prow).astype(jnp.float32)
        stok = (jax.lax.dot_general(tokcol, m0c, dn,
                                    preferred_element_type=jnp.float32)
                + jax.lax.dot_general(tokcol, m1c, dn,
                                      preferred_element_type=jnp.float32))
        swt = (jax.lax.dot_general(w0, m0c, dn,
                                   preferred_element_type=jnp.float32)
               + jax.lax.dot_general(w1, m1c, dn,
                                     preferred_element_type=jnp.float32))
        st_ref[:, CH * ch:CH * (ch + 1)] = stok.astype(jnp.int32)
        sw_ref[:, CH * ch:CH * (ch + 1)] = swt

    tstart = (jax.lax.broadcasted_iota(jnp.int32, (NTILE, 1), 0)
              * G).astype(jnp.float32)
    te_ref[...] = jnp.sum((tstart >= off).astype(jnp.int32),
                          axis=1, keepdims=True) - 1


def _expert_kernel(te_ref, st_ref, acf_ref, sw_ref, yinit_ref,
                   w1_ref, b1_ref, w2_ref, b2_ref, w3_ref, b3_ref,
                   y_ref, x_ref):
    t = pl.program_id(0)

    @pl.when(t == 0)
    def _():
        y_ref[...] = yinit_ref[...]

    for gi in range(G):
        tk = st_ref[G * t + gi]
        base = (tk // 2) * 8
        hi = (tk % 2) == 1
        x8 = acf_ref[pl.ds(base, 8), :]
        x_ref[T2 * gi:T2 * (gi + 1), :] = jnp.where(hi, x8[4:8, :], x8[0:4, :])
    x = x_ref[...]
    h1 = jnp.dot(x, w1_ref[0], preferred_element_type=jnp.float32) + b1_ref[0]
    h2 = jnp.dot(x, w2_ref[0], preferred_element_type=jnp.float32) + b2_ref[0]
    o = jnp.dot(h1 * _sig(h1) * h2, w3_ref[0],
                preferred_element_type=jnp.float32) + b3_ref[0]
    for gi in range(G):
        tk = st_ref[G * t + gi]
        base = (tk // 2) * 8
        hi = (tk % 2) == 1
        wv = sw_ref[pl.ds(G * t + gi, 1), :]
        contrib = wv * o[T2 * gi:T2 * (gi + 1), :]
        zero = jnp.zeros_like(contrib)
        add8 = jnp.concatenate(
            [jnp.where(hi, zero, contrib), jnp.where(hi, contrib, zero)],
            axis=0)
        y_ref[pl.ds(base, 8), :] = y_ref[pl.ds(base, 8), :] + add8


def _shared_kernel(acf_ref, w1_ref, b1_ref, w2_ref, b2_ref, w3_ref, b3_ref,
                   y_ref):
    x = acf_ref[...]
    h1 = jnp.dot(x, w1_ref[...], preferred_element_type=jnp.float32) + b1_ref[...]
    h2 = jnp.dot(x, w2_ref[...], preferred_element_type=jnp.float32) + b2_ref[...]
    y_ref[...] = jnp.dot(h1 * _sig(h1) * h2, w3_ref[...],
                         preferred_element_type=jnp.float32) + b3_ref[...]


def _attn_kernel(cif_ref, y_ref, sm_ref, um_ref, wq_ref, bq_ref,
                 w1_ref, b1_ref, w2_ref, b2_ref, w3_ref, b3_ref, out_ref):
    TOK = NT // 8
    FL = TOK * T2
    q = jnp.dot(cif_ref[...], wq_ref[...],
                preferred_element_type=jnp.float32) + bq_ref[...]
    y = y_ref[...]
    kf = y[:, :C]
    vf = y[:, C:]
    r0 = jax.lax.broadcasted_iota(jnp.int32, (FL, TOK), 0)
    c0 = jax.lax.broadcasted_iota(jnp.int32, (FL, TOK), 1)
    rm = (r0 // T2 == c0).astype(jnp.float32)
    r1 = jax.lax.broadcasted_iota(jnp.int32, (TOK, FL), 0)
    c1 = jax.lax.broadcasted_iota(jnp.int32, (TOK, FL), 1)
    rt = (r1 == c1 // T2).astype(jnp.float32)
    qrep = jnp.dot(rm, q, preferred_element_type=jnp.float32)
    s = jnp.sum(qrep * kf, axis=1, keepdims=True) * (1.0 / math.sqrt(C))
    fl_iota = jax.lax.broadcasted_iota(jnp.int32, (FL, 1), 0)
    lane4 = jax.lax.broadcasted_iota(jnp.int32, (1, T2), 1)
    logits = jnp.zeros((TOK, T2), jnp.float32)
    for j in range(T2):
        ej = (fl_iota % T2 == j).astype(jnp.float32)
        cj = jnp.dot(rt, s * ej, preferred_element_type=jnp.float32)
        logits = logits + cj * (lane4 == j).astype(jnp.float32)
    um = um_ref[...]
    lm = jnp.where(um > 0, logits, NEG)
    mxl = jnp.max(lm, axis=1, keepdims=True)
    ex = jnp.exp(lm - mxl) * um
    den = jnp.sum(ex, axis=1, keepdims=True)
    attn = ex / jnp.where(den == 0, 1.0, den)
    arep = jnp.zeros((FL, 1), jnp.float32)
    for j in range(T2):
        ej = (fl_iota % T2 == j).astype(jnp.float32)
        arep = arep + ej * jnp.dot(rm, attn[:, j:j + 1],
                                   preferred_element_type=jnp.float32)
    unsel = jnp.dot(rt, arep * vf, preferred_element_type=jnp.float32)
    sel = jnp.dot(rt, sm_ref[...] * vf, preferred_element_type=jnp.float32)
    f = jnp.concatenate([sel, unsel], axis=1)
    h1 = jnp.dot(f, w1_ref[...], preferred_element_type=jnp.float32) + b1_ref[...]
    h2 = jnp.dot(f, w2_ref[...], preferred_element_type=jnp.float32) + b2_ref[...]
    out_ref[...] = jnp.dot(h1 * _sig(h1) * h2, w3_ref[...],
                           preferred_element_type=jnp.float32) + b3_ref[...]


@jax.jit
def kernel(traj_feat, traj_len, adj_feat, route_choice_travel_progress,
           route_choice_angle, route_choice_trans_prob,
           route_choice_selected_mask, route_choice_unselected_mask,
           Wtp, btp, Wang, bang, Wtr, btr, Wq, bq, Wg, bg,
           eW1, eb1, eW2, eb2, eW3, eb3, sW1, sb1, sW2, sb2, sW3, sb3,
           fW1, fb1, fW2, fb2, fW3, fb3):
    f32 = jnp.float32
    tp = route_choice_travel_progress[..., None] * Wtp[0] + btp
    cif = jnp.concatenate([traj_feat, tp], axis=-1).reshape(NT, 2 * C)
    ang = route_choice_angle[..., None] * math.pi
    angf = jnp.sin(ang) * Wang[0] + jnp.cos(ang) * Wang[1] + bang
    trf = route_choice_trans_prob[..., None] * Wtr[0] + btr
    acf = jnp.concatenate([adj_feat, angf, trf], axis=-1).reshape(NR, 3 * C)
    mask = (jnp.arange(T1)[None, :] < traj_len[:, None]).astype(f32)
    mask = mask.reshape(NT, 1)

    sds = jax.ShapeDtypeStruct
    vio, te, st, sw = pl.pallas_call(
        _gate_route_kernel,
        out_shape=[sds((1, 1), f32), sds((NTILE, 1), jnp.int32),
                   sds((1, NSLOT), jnp.int32), sds((1, NSLOT), f32)],
    )(cif, Wg, bg.reshape(1, E), mask)
    te_arr = te.reshape(NTILE)
    st_arr = st.reshape(NSLOT)
    sw_arr = sw.reshape(NSLOT, 1)

    y_init = pl.pallas_call(
        _shared_kernel,
        grid=(NR // 128,),
        in_specs=[
            pl.BlockSpec((128, 3 * C), lambda i: (i, 0)),
            pl.BlockSpec((3 * C, 4 * C), lambda i: (0, 0)),
            pl.BlockSpec((1, 4 * C), lambda i: (0, 0)),
            pl.BlockSpec((3 * C, 4 * C), lambda i: (0, 0)),
            pl.BlockSpec((1, 4 * C), lambda i: (0, 0)),
            pl.BlockSpec((4 * C, 2 * C), lambda i: (0, 0)),
            pl.BlockSpec((1, 2 * C), lambda i: (0, 0)),
        ],
        out_specs=pl.BlockSpec((128, 2 * C), lambda i: (i, 0)),
        out_shape=sds((NR, 2 * C), f32),
    )(acf, sW1, sb1.reshape(1, 4 * C), sW2, sb2.reshape(1, 4 * C),
      sW3, sb3.reshape(1, 2 * C))

    eb1r = eb1.reshape(E, 1, 4 * C)
    eb2r = eb2.reshape(E, 1, 4 * C)
    eb3r = eb3.reshape(E, 1, 2 * C)
    y = pl.pallas_call(
        _expert_kernel,
        grid_spec=pltpu.PrefetchScalarGridSpec(
            num_scalar_prefetch=2,
            grid=(NTILE,),
            in_specs=[
                pl.BlockSpec((NR, 3 * C), lambda t, te, st: (0, 0)),
                pl.BlockSpec((NSLOT, 1), lambda t, te, st: (0, 0)),
                pl.BlockSpec((NR, 2 * C), lambda t, te, st: (0, 0)),
                pl.BlockSpec((1, 3 * C, 4 * C), lambda t, te, st: (te[t], 0, 0)),
                pl.BlockSpec((1, 1, 4 * C), lambda t, te, st: (te[t], 0, 0)),
                pl.BlockSpec((1, 3 * C, 4 * C), lambda t, te, st: (te[t], 0, 0)),
                pl.BlockSpec((1, 1, 4 * C), lambda t, te, st: (te[t], 0, 0)),
                pl.BlockSpec((1, 4 * C, 2 * C), lambda t, te, st: (te[t], 0, 0)),
                pl.BlockSpec((1, 1, 2 * C), lambda t, te, st: (te[t], 0, 0)),
            ],
            out_specs=pl.BlockSpec((NR, 2 * C), lambda t, te, st: (0, 0)),
            scratch_shapes=[pltpu.VMEM((G * T2, 3 * C), jnp.float32)],
        ),
        out_shape=sds((NR, 2 * C), f32),
        compiler_params=pltpu.CompilerParams(
            dimension_semantics=("arbitrary",)),
    )(te_arr, st_arr, acf, sw_arr, y_init, eW1, eb1r, eW2, eb2r, eW3, eb3r)

    smask = route_choice_selected_mask.astype(f32).reshape(NR, 1)
    umask = route_choice_unselected_mask.astype(f32).reshape(NT, T2)
    out = pl.pallas_call(
        _attn_kernel,
        grid=(8,),
        in_specs=[
            pl.BlockSpec((NT // 8, 2 * C), lambda i: (i, 0)),
            pl.BlockSpec((NR // 8, 2 * C), lambda i: (i, 0)),
            pl.BlockSpec((NR // 8, 1), lambda i: (i, 0)),
            pl.BlockSpec((NT // 8, T2), lambda i: (i, 0)),
            pl.BlockSpec((2 * C, C), lambda i: (0, 0)),
            pl.BlockSpec((1, C), lambda i: (0, 0)),
            pl.BlockSpec((2 * C, 4 * C), lambda i: (0, 0)),
            pl.BlockSpec((1, 4 * C), lambda i: (0, 0)),
            pl.BlockSpec((2 * C, 4 * C), lambda i: (0, 0)),
            pl.BlockSpec((1, 4 * C), lambda i: (0, 0)),
            pl.BlockSpec((4 * C, C), lambda i: (0, 0)),
            pl.BlockSpec((1, C), lambda i: (0, 0)),
        ],
        out_specs=pl.BlockSpec((NT // 8, C), lambda i: (i, 0)),
        out_shape=sds((NT, C), f32),
    )(cif, y, smask, umask, Wq, bq.reshape(1, C),
      fW1, fb1.reshape(1, 4 * C), fW2, fb2.reshape(1, 4 * C),
      fW3, fb3.reshape(1, C))

    return out.reshape(B, T1, C), vio[0, 0]
